# Initial kernel scaffold; baseline (speedup 1.0000x reference)
import jax
import jax.numpy as jnp
from jax.experimental import pallas as pl


def kernel(x, W_enc, b_enc, W_dec, b_dec):
    raise NotImplementedError("write your pallas kernel here")


# speedup vs baseline: 7.2201x; 7.2201x over previous
import functools

import jax
import jax.numpy as jnp
from jax.experimental import pallas as pl
from jax.experimental.pallas import tpu as pltpu

K = 64


def _encode_body(nj, xb, we, be, bd, out):
    j = pl.program_id(1)
    C = we.shape[0]
    xc = xb[...] - bd[...]
    acc = jax.lax.dot_general(
        xc, we[...], (((1,), (1,)), ((), ())),
        preferred_element_type=jnp.float32)
    acc = jnp.maximum(acc + be[0, pl.ds(j * C, C)][None, :], 0.0)
    out[:, pl.ds(j * C, C)] = acc

    @pl.when(j == nj - 1)
    def _():
        S = out.shape[1]
        CH = min(2048, S)
        nch = S // CH

        def cnt_for(cand):
            def body(c, acc):
                b = jax.lax.bitcast_convert_type(
                    out[:, pl.ds(c * CH, CH)], jnp.int32)
                return acc + jnp.sum((b >= cand).astype(jnp.int32),
                                     axis=1, keepdims=True)
            return jax.lax.fori_loop(0, nch, body, jnp.zeros_like(cand))

        def step(i, t):
            cand = t | (1 << (30 - i))
            return jnp.where(cnt_for(cand) >= K, cand, t)

        t = jax.lax.fori_loop(0, 31, step,
                              jnp.zeros((out.shape[0], 1), jnp.int32))

        def maskbody(c, carry):
            v = out[:, pl.ds(c * CH, CH)]
            b = jax.lax.bitcast_convert_type(v, jnp.int32)
            out[:, pl.ds(c * CH, CH)] = jnp.where(b >= t, v, 0.0)
            return carry

        jax.lax.fori_loop(0, nch, maskbody, 0)


def _decode_body(xe, wd, bd, out):
    j = pl.program_id(1)

    @pl.when(j == 0)
    def _():
        out[...] = jnp.broadcast_to(bd[...], out.shape)

    out[...] += jax.lax.dot_general(
        xe[...], wd[...], (((1,), (1,)), ((), ())),
        preferred_element_type=jnp.float32)


def kernel(x, W_enc, b_enc, W_dec, b_dec):
    N, D = x.shape
    S = W_enc.shape[0]
    R = min(256, N)
    CE = min(1024, S)
    CD = min(2048, S)
    ni, nje, njd = N // R, S // CE, S // CD

    encoded = pl.pallas_call(
        functools.partial(_encode_body, nje),
        grid=(ni, nje),
        in_specs=[
            pl.BlockSpec((R, D), lambda i, j: (i, 0)),
            pl.BlockSpec((CE, D), lambda i, j: (j, 0)),
            pl.BlockSpec((1, S), lambda i, j: (0, 0)),
            pl.BlockSpec((1, D), lambda i, j: (0, 0)),
        ],
        out_specs=pl.BlockSpec((R, S), lambda i, j: (i, 0)),
        out_shape=jax.ShapeDtypeStruct((N, S), jnp.float32),
    )(x, W_enc, b_enc.reshape(1, S), b_dec.reshape(1, D))

    x_hat = pl.pallas_call(
        _decode_body,
        grid=(ni, njd),
        in_specs=[
            pl.BlockSpec((R, CD), lambda i, j: (i, j)),
            pl.BlockSpec((D, CD), lambda i, j: (0, j)),
            pl.BlockSpec((1, D), lambda i, j: (0, 0)),
        ],
        out_specs=pl.BlockSpec((R, D), lambda i, j: (i, 0)),
        out_shape=jax.ShapeDtypeStruct((N, D), jnp.float32),
    )(encoded, W_dec, b_dec.reshape(1, D))
    return x_hat
